# baseline (speedup 1.0000x reference)
import functools

import jax
import jax.numpy as jnp
from jax import lax
from jax.experimental import pallas as pl
from jax.experimental.pallas import tpu as pltpu
from jax.experimental.pallas import tpu_sc as plsc

_NUM_CORES = 2
_NUM_SUBCORES = 16
_NUM_WORKERS = _NUM_CORES * _NUM_SUBCORES

_GROUP = 512
_LANES = 16


def _make_gather(fields: int, batch: int, embed: int):
  assert embed == 32 and batch % (128 * 4) == 0
  n_groups = fields * batch // _GROUP
  groups_per_worker = n_groups // _NUM_WORKERS
  assert groups_per_worker * _NUM_WORKERS == n_groups
  bt_per_field = batch // 128
  gt_per_field = bt_per_field // 4

  mesh = plsc.VectorSubcoreMesh(
      core_axis_name="c", subcore_axis_name="s")

  @functools.partial(
      pl.kernel,
      out_type=jax.ShapeDtypeStruct(
          (fields, embed // 8, batch // 128, 8, 128), jnp.float32),
      mesh=mesh,
      scratch_types=[
          pltpu.VMEM((groups_per_worker, _GROUP), jnp.int32),
          pltpu.VMEM((_GROUP, embed), jnp.float32),
          pltpu.VMEM((_GROUP, embed), jnp.float32),
          pltpu.VMEM((4, 4, 8, 128), jnp.float32),
          pltpu.VMEM((4, 4, 8, 128), jnp.float32),
          pltpu.SemaphoreType.DMA,
          pltpu.SemaphoreType.DMA,
          pltpu.SemaphoreType.DMA((4,)),
          pltpu.SemaphoreType.DMA((4,)),
      ],
      compiler_params=pltpu.CompilerParams(
          use_tc_tiling_on_sc=False, needs_layout_passes=False),
  )
  def gather_kernel(idx_hbm, table_hbm, out_hbm, idx_v, rows0, rows1,
                    tst0, tst1, gsem0, gsem1, osem0, osem1):
    wid = lax.axis_index("s") * _NUM_CORES + lax.axis_index("c")
    g0 = wid * groups_per_worker
    pltpu.sync_copy(idx_hbm.at[wid], idx_v)

    def fire_gather(g, rows, gsem):
      pltpu.async_copy(table_hbm.at[idx_v.at[g]], rows, gsem)

    fire_gather(0, rows0, gsem0)
    fire_gather(1, rows1, gsem1)

    iota = jax.lax.iota(jnp.int32, _LANES)

    def transpose_group(rows, tst):
      for bt in range(4):
        rbase = iota + bt * 128
        for e in range(embed):
          cidx = jnp.full((_LANES,), e, dtype=jnp.int32)
          for gg in range(128 // _LANES):
            v = plsc.load_gather(rows, [rbase + gg * _LANES, cidx])
            tst[e // 8, bt, e % 8, pl.ds(gg * _LANES, _LANES)] = v

    def out_slices(g, tst, osem):
      gg = g0 + g
      f = gg // gt_per_field
      bt0 = (gg % gt_per_field) * 4
      return [
          pltpu.make_async_copy(
              tst.at[et], out_hbm.at[f, et, pl.ds(bt0, 4)], osem.at[et])
          for et in range(4)
      ]

    def process(g, rows, gsem, tst, osem):
      pltpu.make_async_copy(table_hbm.at[idx_v.at[g]], rows, gsem).wait()

      @pl.when(g >= 2)
      def _():
        for c in out_slices(g - 2, tst, osem):
          c.wait()

      transpose_group(rows, tst)

      for c in out_slices(g, tst, osem):
        c.start()

      @pl.when(g + 2 < groups_per_worker)
      def _():
        fire_gather(g + 2, rows, gsem)

    def outer(i, carry):
      process(2 * i, rows0, gsem0, tst0, osem0)
      process(2 * i + 1, rows1, gsem1, tst1, osem1)
      return carry

    lax.fori_loop(0, groups_per_worker // 2, outer, 0)

    for c in out_slices(groups_per_worker - 2, tst0, osem0):
      c.wait()
    for c in out_slices(groups_per_worker - 1, tst1, osem1):
      c.wait()

  return gather_kernel


def kernel(x, table):
  batch, fields = x.shape
  vocab, embed = table.shape
  idx = x.T.reshape(_NUM_WORKERS, (batch * fields) // (_NUM_WORKERS * _GROUP),
                    _GROUP)
  idx = idx.astype(jnp.int32)
  out5 = _make_gather(fields, batch, embed)(idx, table)
  out5 = out5.transpose(2, 4, 0, 1, 3)
  return out5.reshape(batch, fields, embed)


# speedup vs baseline: 1.0881x; 1.0881x over previous
import functools

import jax
import jax.numpy as jnp
from jax import lax
from jax.experimental import pallas as pl
from jax.experimental.pallas import tpu as pltpu
from jax.experimental.pallas import tpu_sc as plsc

_NUM_CORES = 2
_NUM_SUBCORES = 16
_NUM_WORKERS = _NUM_CORES * _NUM_SUBCORES

_CHUNK = 512
_NBUF = 4


def _make_gather(num_rows: int, embed: int):
  assert num_rows % (_NUM_WORKERS * _CHUNK) == 0
  rows_per_worker = num_rows // _NUM_WORKERS
  n_chunks = rows_per_worker // _CHUNK
  ahead = _NBUF - 1
  assert n_chunks >= ahead

  mesh = plsc.VectorSubcoreMesh(
      core_axis_name="c", subcore_axis_name="s")

  @functools.partial(
      pl.kernel,
      out_type=jax.ShapeDtypeStruct((num_rows, embed), jnp.float32),
      mesh=mesh,
      scratch_types=[
          pltpu.VMEM((n_chunks, _CHUNK), jnp.int32),
          pltpu.VMEM((_NBUF, _CHUNK, embed), jnp.float32),
          pltpu.SemaphoreType.DMA((_NBUF,)),
          pltpu.SemaphoreType.DMA((_NBUF,)),
      ],
      compiler_params=pltpu.CompilerParams(use_tc_tiling_on_sc=False),
  )
  def gather_kernel(idx_hbm, table_hbm, out_hbm, idx_v, rows_v, gsem, osem):
    wid = lax.axis_index("s") * _NUM_CORES + lax.axis_index("c")
    base = wid * rows_per_worker
    pltpu.sync_copy(idx_hbm.at[wid], idx_v)

    for j in range(ahead):
      pltpu.async_copy(table_hbm.at[idx_v.at[j]], rows_v.at[j], gsem.at[j])

    def step(j, carry):
      b = lax.rem(j, _NBUF)
      bp = lax.rem(j + _NBUF - 1, _NBUF)
      pltpu.make_async_copy(
          table_hbm.at[idx_v.at[j]], rows_v.at[b], gsem.at[b]).wait()

      @pl.when(j >= 1)
      def _():
        pltpu.make_async_copy(
            rows_v.at[bp],
            out_hbm.at[pl.ds(base + (j - 1) * _CHUNK, _CHUNK)],
            osem.at[bp]).wait()

      @pl.when(j + ahead < n_chunks)
      def _():
        pltpu.async_copy(
            table_hbm.at[idx_v.at[j + ahead]], rows_v.at[bp], gsem.at[bp])

      pltpu.async_copy(
          rows_v.at[b], out_hbm.at[pl.ds(base + j * _CHUNK, _CHUNK)],
          osem.at[b])
      return carry

    lax.fori_loop(0, n_chunks, step, 0)

    bl = (n_chunks - 1) % _NBUF
    pltpu.make_async_copy(
        rows_v.at[bl],
        out_hbm.at[pl.ds(base + (n_chunks - 1) * _CHUNK, _CHUNK)],
        osem.at[bl]).wait()

  return gather_kernel


def _make_relayout(fields: int, batch: int, embed: int):
  bt_blk = 32
  rows_blk = bt_blk * 128

  def body(in_ref, out_ref):
    x = in_ref[...]
    x = x.reshape(bt_blk, 128, embed)
    y = jnp.swapaxes(x, 1, 2)
    y = y.reshape(bt_blk, embed // 8, 8, 128)
    out_ref[0] = jnp.transpose(y, (1, 0, 2, 3))

  n_q = batch // rows_blk
  return pl.pallas_call(
      body,
      grid=(fields, n_q),
      in_specs=[
          pl.BlockSpec((rows_blk, embed), lambda f, q: (f * n_q + q, 0))
      ],
      out_specs=pl.BlockSpec(
          (1, embed // 8, bt_blk, 8, 128), lambda f, q: (f, 0, q, 0, 0)),
      out_shape=jax.ShapeDtypeStruct(
          (fields, embed // 8, batch // 128, 8, 128), jnp.float32),
  )


def kernel(x, table):
  batch, fields = x.shape
  vocab, embed = table.shape
  num_rows = batch * fields
  idx = x.T.reshape(_NUM_WORKERS, num_rows // (_NUM_WORKERS * _CHUNK), _CHUNK)
  idx = idx.astype(jnp.int32)
  flat = _make_gather(num_rows, embed)(idx, table)
  out5 = _make_relayout(fields, batch, embed)(flat)
  out5 = out5.transpose(2, 4, 0, 1, 3)
  return out5.reshape(batch, fields, embed)
